# Initial kernel scaffold; baseline (speedup 1.0000x reference)
import jax
import jax.numpy as jnp
from jax.experimental import pallas as pl


def kernel(x, w1f, b1p, w2f, b2p):
    raise NotImplementedError("write your pallas kernel here")


# speedup vs baseline: 2.5006x; 2.5006x over previous
from functools import partial

import jax
import jax.numpy as jnp
from jax import lax
from jax.experimental import pallas as pl
from jax.experimental.pallas import tpu as pltpu

_EPS = 1e-5
_VMEM_LIMIT = 60 * 1024 * 1024
_TILE_D = 4
_CIN, _MID, _COUT = 32, 32, 64
_LANE_PAD = 128


def _conv_stats_kernel(x_hbm, w_ref, b_ref, *args, tile_d, norm_in):
    if norm_in:
        scale_ref, shift_ref, y_ref, stats_ref, xs_ref, sem = args
    else:
        y_ref, stats_ref, xs_ref, sem = args

    n = pl.program_id(0)
    i = pl.program_id(1)
    d0 = pl.multiple_of(i * tile_d, tile_d)

    _, _, H, W, cout = y_ref.shape
    cin = xs_ref.shape[-1]

    cp = pltpu.make_async_copy(x_hbm.at[n, pl.ds(d0, tile_d + 2)], xs_ref, sem)
    cp.start()
    cp.wait()

    xb = xs_ref[...]
    if norm_in:
        xf = xb.astype(jnp.float32) * scale_ref[...] + shift_ref[...]
        xb = jnp.maximum(xf, 0.0).astype(jnp.bfloat16)

    acc = None
    for kd in range(3):
        for kh in range(3):
            sl = xb[kd:kd + tile_d, kh:kh + H]
            patch = jnp.concatenate(
                [sl[:, :, 0:W], sl[:, :, 1:W + 1], sl[:, :, 2:W + 2]],
                axis=-1,
            ).reshape(tile_d * H * W, 3 * cin)
            part = jnp.dot(patch, w_ref[kd, kh], preferred_element_type=jnp.float32)
            acc = part if acc is None else acc + part

    acc = acc + b_ref[...]
    y_ref[...] = acc.reshape(y_ref.shape).astype(y_ref.dtype)

    s1 = jnp.sum(acc, axis=0, keepdims=True)
    s2 = jnp.sum(acc * acc, axis=0, keepdims=True)
    pad = jnp.zeros((6, cout), jnp.float32)
    stats_ref[...] = jnp.concatenate([s1, s2, pad], axis=0).reshape(stats_ref.shape)


def _conv_block(x_pad, w, b, scale, shift, tile_d):
    N, Dp2, Hp2, Wp2, cin = x_pad.shape
    D, H, W = Dp2 - 2, Hp2 - 2, Wp2 - 2
    cout = w.shape[-1]
    n_d = D // tile_d
    norm_in = scale is not None

    in_specs = [
        pl.BlockSpec(memory_space=pl.ANY),
        pl.BlockSpec((3, 3, 3 * cin, cout), lambda n, i: (0, 0, 0, 0)),
        pl.BlockSpec((1, cout), lambda n, i: (0, 0)),
    ]
    inputs = [x_pad, w, b]
    if norm_in:
        in_specs += [pl.BlockSpec((1, 1, 1, cin), lambda n, i: (n, 0, 0, 0))] * 2
        inputs += [scale.reshape(N, 1, 1, cin), shift.reshape(N, 1, 1, cin)]

    y, stats = pl.pallas_call(
        partial(_conv_stats_kernel, tile_d=tile_d, norm_in=norm_in),
        grid=(N, n_d),
        in_specs=in_specs,
        out_specs=(
            pl.BlockSpec((1, tile_d, H, W, cout), lambda n, i: (n, i, 0, 0, 0)),
            pl.BlockSpec((1, 1, 8, cout), lambda n, i: (n, i, 0, 0)),
        ),
        out_shape=(
            jax.ShapeDtypeStruct((N, D, H, W, cout), jnp.bfloat16),
            jax.ShapeDtypeStruct((N, n_d, 8, cout), jnp.float32),
        ),
        scratch_shapes=[
            pltpu.VMEM((tile_d + 2, Hp2, Wp2, cin), x_pad.dtype),
            pltpu.SemaphoreType.DMA,
        ],
        compiler_params=pltpu.CompilerParams(
            dimension_semantics=("parallel", "parallel"),
            vmem_limit_bytes=_VMEM_LIMIT,
        ),
    )(*inputs)

    cnt = float(D * H * W)
    mean = stats[:, :, 0, :].sum(axis=1) / cnt
    var = jnp.maximum(stats[:, :, 1, :].sum(axis=1) / cnt - mean * mean, 0.0)
    sc = lax.rsqrt(var + _EPS)
    return y, sc, -mean * sc


def _norm_relu_out_kernel(y_ref, s_ref, t_ref, o_ref):
    y = y_ref[...].astype(jnp.float32)
    o_ref[...] = jnp.maximum(y * s_ref[...] + t_ref[...], 0.0)


def kernel(x, w1f, b1p, w2f, b2p):
    N, cin, D, H, W = x.shape
    lp = _LANE_PAD

    w1 = w1f.reshape(3, 3, 3, lp, lp)[:, :, :, :_CIN, :_MID]
    w1 = w1.reshape(3, 3, 3 * _CIN, _MID).astype(jnp.bfloat16)
    w2 = w2f.reshape(3, 3, 3, lp, lp)[:, :, :, :_MID, :_COUT]
    w2 = w2.reshape(3, 3, 3 * _MID, _COUT).astype(jnp.bfloat16)
    b1 = b1p[:, :_MID]
    b2 = b2p[:, :_COUT]

    xt = jnp.transpose(x, (0, 2, 3, 4, 1)).astype(jnp.bfloat16)
    xp = jnp.pad(xt, ((0, 0), (1, 1), (1, 1), (1, 1), (0, 0)))

    y1, sc1, sh1 = _conv_block(xp, w1, b1, None, None, _TILE_D)
    y1p = jnp.pad(y1, ((0, 0), (1, 1), (1, 1), (1, 1), (0, 0)),
                  constant_values=-jnp.inf)
    y2, sc2, sh2 = _conv_block(y1p, w2, b2, sc1, sh1, _TILE_D)

    n_d = D // _TILE_D
    out = pl.pallas_call(
        _norm_relu_out_kernel,
        grid=(N, n_d),
        in_specs=[
            pl.BlockSpec((1, _TILE_D, H, W, _COUT), lambda n, i: (n, i, 0, 0, 0)),
            pl.BlockSpec((1, 1, 1, 1, _COUT), lambda n, i: (n, 0, 0, 0, 0)),
            pl.BlockSpec((1, 1, 1, 1, _COUT), lambda n, i: (n, 0, 0, 0, 0)),
        ],
        out_specs=pl.BlockSpec((1, _TILE_D, H, W, _COUT), lambda n, i: (n, i, 0, 0, 0)),
        out_shape=jax.ShapeDtypeStruct((N, D, H, W, _COUT), jnp.float32),
        compiler_params=pltpu.CompilerParams(
            dimension_semantics=("parallel", "parallel"),
            vmem_limit_bytes=_VMEM_LIMIT,
        ),
    )(y2, sc2.reshape(N, 1, 1, 1, _COUT), sh2.reshape(N, 1, 1, 1, _COUT))

    return jnp.transpose(out, (0, 4, 1, 2, 3))


# speedup vs baseline: 2.6809x; 1.0721x over previous
from functools import partial

import jax
import jax.numpy as jnp
from jax import lax
from jax.experimental import pallas as pl
from jax.experimental.pallas import tpu as pltpu

_EPS = 1e-5
_VMEM_LIMIT = 60 * 1024 * 1024
_TILE_D = 4
_CIN, _MID, _COUT = 32, 32, 64
_LANE_PAD = 128


def _conv_stats_kernel(x_hbm, w_ref, b_ref, *args, tile_d, norm_in):
    if norm_in:
        scale_ref, shift_ref, y_ref, stats_ref, xs_ref, sem = args
    else:
        y_ref, stats_ref, xs_ref, sem = args

    n = pl.program_id(0)
    i = pl.program_id(1)
    d0 = pl.multiple_of(i * tile_d, tile_d)

    _, _, H, W, cout = y_ref.shape
    cin = xs_ref.shape[-1]

    cp = pltpu.make_async_copy(x_hbm.at[n, pl.ds(d0, tile_d + 2)], xs_ref, sem)
    cp.start()
    cp.wait()

    xb = xs_ref[...]
    if norm_in:
        xf = xb.astype(jnp.float32) * scale_ref[...] + shift_ref[...]
        xb = jnp.maximum(xf, 0.0).astype(jnp.bfloat16)

    acc = None
    for kd in range(3):
        for kh in range(3):
            sl = xb[kd:kd + tile_d, kh:kh + H]
            patch = jnp.concatenate(
                [sl[:, :, 0:W], sl[:, :, 1:W + 1], sl[:, :, 2:W + 2]],
                axis=-1,
            ).reshape(tile_d * H * W, 3 * cin)
            part = jnp.dot(patch, w_ref[kd, kh], preferred_element_type=jnp.float32)
            acc = part if acc is None else acc + part

    acc = acc + b_ref[...]
    y_ref[...] = acc.reshape(y_ref.shape).astype(y_ref.dtype)

    s1 = jnp.sum(acc, axis=0, keepdims=True)
    s2 = jnp.sum(acc * acc, axis=0, keepdims=True)
    pad = jnp.zeros((6, cout), jnp.float32)
    stats_ref[...] = jnp.concatenate([s1, s2, pad], axis=0).reshape(stats_ref.shape)


def _conv_block(x_pad, w, b, scale, shift, tile_d):
    N, Dp2, Hp2, Wp2, cin = x_pad.shape
    D, H, W = Dp2 - 2, Hp2 - 2, Wp2 - 2
    cout = w.shape[-1]
    n_d = D // tile_d
    norm_in = scale is not None

    in_specs = [
        pl.BlockSpec(memory_space=pl.ANY),
        pl.BlockSpec((3, 3, 3 * cin, cout), lambda n, i: (0, 0, 0, 0)),
        pl.BlockSpec((1, cout), lambda n, i: (0, 0)),
    ]
    inputs = [x_pad, w, b]
    if norm_in:
        in_specs += [pl.BlockSpec((1, 1, 1, cin), lambda n, i: (n, 0, 0, 0))] * 2
        inputs += [scale.reshape(N, 1, 1, cin), shift.reshape(N, 1, 1, cin)]

    y, stats = pl.pallas_call(
        partial(_conv_stats_kernel, tile_d=tile_d, norm_in=norm_in),
        grid=(N, n_d),
        in_specs=in_specs,
        out_specs=(
            pl.BlockSpec((1, tile_d, H, W, cout), lambda n, i: (n, i, 0, 0, 0)),
            pl.BlockSpec((1, 1, 8, cout), lambda n, i: (n, i, 0, 0)),
        ),
        out_shape=(
            jax.ShapeDtypeStruct((N, D, H, W, cout), jnp.bfloat16),
            jax.ShapeDtypeStruct((N, n_d, 8, cout), jnp.float32),
        ),
        scratch_shapes=[
            pltpu.VMEM((tile_d + 2, Hp2, Wp2, cin), x_pad.dtype),
            pltpu.SemaphoreType.DMA,
        ],
        compiler_params=pltpu.CompilerParams(
            dimension_semantics=("parallel", "parallel"),
            vmem_limit_bytes=_VMEM_LIMIT,
        ),
    )(*inputs)

    cnt = float(D * H * W)
    mean = stats[:, :, 0, :].sum(axis=1) / cnt
    var = jnp.maximum(stats[:, :, 1, :].sum(axis=1) / cnt - mean * mean, 0.0)
    sc = lax.rsqrt(var + _EPS)
    return y, sc, -mean * sc


def _norm_relu_out_kernel(y_ref, s_ref, t_ref, o_ref):
    y = y_ref[0].astype(jnp.float32)
    yn = jnp.maximum(y * s_ref[0, 0, 0] + t_ref[0, 0, 0], 0.0)
    o_ref[...] = jnp.transpose(yn, (3, 0, 1, 2))[None]


def kernel(x, w1f, b1p, w2f, b2p):
    N, cin, D, H, W = x.shape
    lp = _LANE_PAD

    w1 = w1f.reshape(3, 3, 3, lp, lp)[:, :, :, :_CIN, :_MID]
    w1 = w1.reshape(3, 3, 3 * _CIN, _MID).astype(jnp.bfloat16)
    w2 = w2f.reshape(3, 3, 3, lp, lp)[:, :, :, :_MID, :_COUT]
    w2 = w2.reshape(3, 3, 3 * _MID, _COUT).astype(jnp.bfloat16)
    b1 = b1p[:, :_MID]
    b2 = b2p[:, :_COUT]

    xt = jnp.transpose(x, (0, 2, 3, 4, 1)).astype(jnp.bfloat16)
    xp = jnp.pad(xt, ((0, 0), (1, 1), (1, 1), (1, 1), (0, 0)))

    y1, sc1, sh1 = _conv_block(xp, w1, b1, None, None, _TILE_D)
    y1p = jnp.pad(y1, ((0, 0), (1, 1), (1, 1), (1, 1), (0, 0)),
                  constant_values=-jnp.inf)
    y2, sc2, sh2 = _conv_block(y1p, w2, b2, sc1, sh1, _TILE_D)

    n_d = D // _TILE_D
    out = pl.pallas_call(
        _norm_relu_out_kernel,
        grid=(N, n_d),
        in_specs=[
            pl.BlockSpec((1, _TILE_D, H, W, _COUT), lambda n, i: (n, i, 0, 0, 0)),
            pl.BlockSpec((1, 1, 1, 1, _COUT), lambda n, i: (n, 0, 0, 0, 0)),
            pl.BlockSpec((1, 1, 1, 1, _COUT), lambda n, i: (n, 0, 0, 0, 0)),
        ],
        out_specs=pl.BlockSpec((1, _COUT, _TILE_D, H, W), lambda n, i: (n, 0, i, 0, 0)),
        out_shape=jax.ShapeDtypeStruct((N, _COUT, D, H, W), jnp.float32),
        compiler_params=pltpu.CompilerParams(
            dimension_semantics=("parallel", "parallel"),
            vmem_limit_bytes=_VMEM_LIMIT,
        ),
    )(y2, sc2.reshape(N, 1, 1, 1, _COUT), sh2.reshape(N, 1, 1, 1, _COUT))

    return out


# speedup vs baseline: 2.6824x; 1.0005x over previous
from functools import partial

import jax
import jax.numpy as jnp
from jax import lax
from jax.experimental import pallas as pl
from jax.experimental.pallas import tpu as pltpu

_EPS = 1e-5
_VMEM_LIMIT = 60 * 1024 * 1024
_TILE_D = 4
_CIN, _MID, _COUT = 32, 32, 64
_LANE_PAD = 128


def _conv_stats_kernel(x_hbm, w_ref, b_ref, *args, tile_d, norm_in):
    if norm_in:
        scale_ref, shift_ref, y_ref, stats_ref, xs_ref, sem = args
    else:
        y_ref, stats_ref, xs_ref, sem = args

    n = pl.program_id(0)
    i = pl.program_id(1)
    d0 = pl.multiple_of(i * tile_d, tile_d)

    _, _, H, W, cout = y_ref.shape
    cin = xs_ref.shape[-1]

    cp = pltpu.make_async_copy(x_hbm.at[n, pl.ds(d0, tile_d + 2)], xs_ref, sem)
    cp.start()
    cp.wait()

    xb = xs_ref[...]
    if norm_in:
        xf = xb.astype(jnp.float32) * scale_ref[...] + shift_ref[...]
        xb = jnp.maximum(xf, 0.0).astype(jnp.bfloat16)

    acc = None
    for kd in range(3):
        for kh in range(3):
            sl = xb[kd:kd + tile_d, kh:kh + H]
            patch = jnp.concatenate(
                [sl[:, :, 0:W], sl[:, :, 1:W + 1], sl[:, :, 2:W + 2]],
                axis=-1,
            ).reshape(tile_d * H * W, 3 * cin)
            part = jnp.dot(patch, w_ref[kd, kh], preferred_element_type=jnp.float32)
            acc = part if acc is None else acc + part

    acc = acc + b_ref[...]
    y_ref[...] = acc.reshape(y_ref.shape).astype(y_ref.dtype)

    s1 = jnp.sum(acc, axis=0, keepdims=True)
    s2 = jnp.sum(acc * acc, axis=0, keepdims=True)
    pad = jnp.zeros((6, cout), jnp.float32)
    stats_ref[...] = jnp.concatenate([s1, s2, pad], axis=0).reshape(stats_ref.shape)


def _conv_block(x_pad, w, b, scale, shift, tile_d):
    N, Dp2, Hp2, Wp2, cin = x_pad.shape
    D, H, W = Dp2 - 2, Hp2 - 2, Wp2 - 2
    cout = w.shape[-1]
    n_d = D // tile_d
    norm_in = scale is not None

    in_specs = [
        pl.BlockSpec(memory_space=pl.ANY),
        pl.BlockSpec((3, 3, 3 * cin, cout), lambda n, i: (0, 0, 0, 0)),
        pl.BlockSpec((1, cout), lambda n, i: (0, 0)),
    ]
    inputs = [x_pad, w, b]
    if norm_in:
        in_specs += [pl.BlockSpec((1, 1, 1, cin), lambda n, i: (n, 0, 0, 0))] * 2
        inputs += [scale.reshape(N, 1, 1, cin), shift.reshape(N, 1, 1, cin)]

    y, stats = pl.pallas_call(
        partial(_conv_stats_kernel, tile_d=tile_d, norm_in=norm_in),
        grid=(N, n_d),
        in_specs=in_specs,
        out_specs=(
            pl.BlockSpec((1, tile_d, H, W, cout), lambda n, i: (n, i, 0, 0, 0)),
            pl.BlockSpec((1, 1, 8, cout), lambda n, i: (n, i, 0, 0)),
        ),
        out_shape=(
            jax.ShapeDtypeStruct((N, D, H, W, cout), jnp.bfloat16),
            jax.ShapeDtypeStruct((N, n_d, 8, cout), jnp.float32),
        ),
        scratch_shapes=[
            pltpu.VMEM((tile_d + 2, Hp2, Wp2, cin), x_pad.dtype),
            pltpu.SemaphoreType.DMA,
        ],
        compiler_params=pltpu.CompilerParams(
            dimension_semantics=("parallel", "parallel"),
            vmem_limit_bytes=_VMEM_LIMIT,
        ),
    )(*inputs)

    cnt = float(D * H * W)
    mean = stats[:, :, 0, :].sum(axis=1) / cnt
    var = jnp.maximum(stats[:, :, 1, :].sum(axis=1) / cnt - mean * mean, 0.0)
    sc = lax.rsqrt(var + _EPS)
    return y, sc, -mean * sc


def _to_ndhwc_kernel(x_ref, o_ref):
    x = x_ref[0].astype(jnp.bfloat16)
    o_ref[...] = jnp.transpose(x, (1, 2, 3, 0))[None]


def _norm_relu_out_kernel(y_ref, s_ref, t_ref, o_ref):
    y = y_ref[0].astype(jnp.float32)
    yn = jnp.maximum(y * s_ref[0, 0, 0] + t_ref[0, 0, 0], 0.0)
    o_ref[...] = jnp.transpose(yn, (3, 0, 1, 2))[None]


def kernel(x, w1f, b1p, w2f, b2p):
    N, cin, D, H, W = x.shape
    lp = _LANE_PAD

    w1 = w1f.reshape(3, 3, 3, lp, lp)[:, :, :, :_CIN, :_MID]
    w1 = w1.reshape(3, 3, 3 * _CIN, _MID).astype(jnp.bfloat16)
    w2 = w2f.reshape(3, 3, 3, lp, lp)[:, :, :, :_MID, :_COUT]
    w2 = w2.reshape(3, 3, 3 * _MID, _COUT).astype(jnp.bfloat16)
    b1 = b1p[:, :_MID]
    b2 = b2p[:, :_COUT]

    n_d0 = D // _TILE_D
    xt = pl.pallas_call(
        _to_ndhwc_kernel,
        grid=(N, n_d0),
        in_specs=[pl.BlockSpec((1, cin, _TILE_D, H, W), lambda n, i: (n, 0, i, 0, 0))],
        out_specs=pl.BlockSpec((1, _TILE_D, H, W, cin), lambda n, i: (n, i, 0, 0, 0)),
        out_shape=jax.ShapeDtypeStruct((N, D, H, W, cin), jnp.bfloat16),
        compiler_params=pltpu.CompilerParams(
            dimension_semantics=("parallel", "parallel"),
            vmem_limit_bytes=_VMEM_LIMIT,
        ),
    )(x)
    xp = jnp.pad(xt, ((0, 0), (1, 1), (1, 1), (1, 1), (0, 0)))

    y1, sc1, sh1 = _conv_block(xp, w1, b1, None, None, _TILE_D)
    y1p = jnp.pad(y1, ((0, 0), (1, 1), (1, 1), (1, 1), (0, 0)),
                  constant_values=-jnp.inf)
    y2, sc2, sh2 = _conv_block(y1p, w2, b2, sc1, sh1, _TILE_D)

    n_d = D // _TILE_D
    out = pl.pallas_call(
        _norm_relu_out_kernel,
        grid=(N, n_d),
        in_specs=[
            pl.BlockSpec((1, _TILE_D, H, W, _COUT), lambda n, i: (n, i, 0, 0, 0)),
            pl.BlockSpec((1, 1, 1, 1, _COUT), lambda n, i: (n, 0, 0, 0, 0)),
            pl.BlockSpec((1, 1, 1, 1, _COUT), lambda n, i: (n, 0, 0, 0, 0)),
        ],
        out_specs=pl.BlockSpec((1, _COUT, _TILE_D, H, W), lambda n, i: (n, 0, i, 0, 0)),
        out_shape=jax.ShapeDtypeStruct((N, _COUT, D, H, W), jnp.float32),
        compiler_params=pltpu.CompilerParams(
            dimension_semantics=("parallel", "parallel"),
            vmem_limit_bytes=_VMEM_LIMIT,
        ),
    )(y2, sc2.reshape(N, 1, 1, 1, _COUT), sh2.reshape(N, 1, 1, 1, _COUT))

    return out
